# baseline (speedup 1.0000x reference)
import functools

import jax
import jax.numpy as jnp
from jax import lax
from jax.experimental import pallas as pl
from jax.experimental.pallas import tpu as pltpu
from jax.experimental.pallas import tpu_sc as plsc

_CHUNK = 1024

_mesh = plsc.VectorSubcoreMesh(core_axis_name="c", subcore_axis_name="s")


@functools.partial(
    pl.kernel,
    mesh=_mesh,
    scratch_types=[pltpu.VMEM((_CHUNK,), jnp.float32)],
)
def _fill(x_ref, buf):
    wid = lax.axis_index("s") * 2 + lax.axis_index("c")
    for i in range(_CHUNK // 16):
        buf[pl.ds(16 * i, 16)] = jnp.full((16,), -1.0, jnp.float32)

    @pl.when(wid < 24)
    def _():
        b = wid // 12
        r = (wid % 12) // 4
        q = wid % 4
        pltpu.sync_copy(buf, x_ref.at[b, r, pl.ds(q * _CHUNK, _CHUNK)])


def kernel(x):
    xr = jax.new_ref(x)
    _fill(xr)
    return xr[...]


# speedup vs baseline: 1.0980x; 1.0980x over previous
import jax
import jax.numpy as jnp
from jax import lax
from jax.experimental import pallas as pl

_BLK = 512


def _body(x_ref, o_ref):
    j = pl.program_id(1)

    @pl.when(j == 0)
    def _():
        v = x_ref[...]
        row = lax.broadcasted_iota(jnp.int32, v.shape, 1)
        o_ref[...] = jnp.where(row < 3, jnp.float32(-1.0), v)

    @pl.when(j != 0)
    def _():
        o_ref[...] = x_ref[...]


def kernel(x):
    b, r, c = x.shape
    return pl.pallas_call(
        _body,
        grid=(b, r // _BLK),
        in_specs=[pl.BlockSpec((1, _BLK, c), lambda i, j: (i, j, 0))],
        out_specs=pl.BlockSpec((1, _BLK, c), lambda i, j: (i, j, 0)),
        out_shape=jax.ShapeDtypeStruct(x.shape, x.dtype),
    )(x)


# speedup vs baseline: 1.1053x; 1.0067x over previous
import jax
import jax.numpy as jnp
from jax import lax
from jax.experimental import pallas as pl

_BLK = 768


def _body(x_ref, o_ref):
    j = pl.program_id(1)

    @pl.when(j == 0)
    def _():
        v = x_ref[...]
        row = lax.broadcasted_iota(jnp.int32, v.shape, 1)
        o_ref[...] = jnp.where(row < 3, jnp.float32(-1.0), v)

    @pl.when(j != 0)
    def _():
        o_ref[...] = x_ref[...]


def kernel(x):
    b, r, c = x.shape
    return pl.pallas_call(
        _body,
        grid=(b, pl.cdiv(r, _BLK)),
        in_specs=[pl.BlockSpec((1, _BLK, c), lambda i, j: (i, j, 0))],
        out_specs=pl.BlockSpec((1, _BLK, c), lambda i, j: (i, j, 0)),
        out_shape=jax.ShapeDtypeStruct(x.shape, x.dtype),
    )(x)


# speedup vs baseline: 1.1054x; 1.0001x over previous
import jax
import jax.numpy as jnp
from jax import lax
from jax.experimental import pallas as pl
from jax.experimental.pallas import tpu as pltpu

_BLK = 960


def _body(x_ref, o_ref):
    j = pl.program_id(1)

    @pl.when(j == 0)
    def _():
        v = x_ref[...]
        row = lax.broadcasted_iota(jnp.int32, v.shape, 1)
        o_ref[...] = jnp.where(row < 3, jnp.float32(-1.0), v)

    @pl.when(j != 0)
    def _():
        o_ref[...] = x_ref[...]


def kernel(x):
    b, r, c = x.shape
    return pl.pallas_call(
        _body,
        grid=(b, pl.cdiv(r, _BLK)),
        in_specs=[pl.BlockSpec((1, _BLK, c), lambda i, j: (i, j, 0))],
        out_specs=pl.BlockSpec((1, _BLK, c), lambda i, j: (i, j, 0)),
        out_shape=jax.ShapeDtypeStruct(x.shape, x.dtype),
        compiler_params=pltpu.CompilerParams(vmem_limit_bytes=100 * 1024 * 1024),
    )(x)
